# baseline (speedup 1.0000x reference)
import dataclasses
import functools

import jax
import jax.numpy as jnp
from jax import lax
from jax.experimental import pallas as pl
from jax.experimental.pallas import tpu as pltpu
from jax.experimental.pallas import tpu_sc as plsc

N = 10000
E = 320000
H = 128
HH = H // 2
G = 32
VOC = 100
NC = 2
NS = 16
NW = NC * NS
BB = 80
EWD = E // NW
NBD = EWD // BB
EWL = E // NS
NBL = EWL // BB
RPT = 624
BLK = 1000

_mesh = plsc.VectorSubcoreMesh(core_axis_name="c", subcore_axis_name="s")

_sc_params = pltpu.CompilerParams()
if "needs_layout_passes" in pltpu.CompilerParams.__dataclass_fields__:
    _sc_params = dataclasses.replace(_sc_params, needs_layout_passes=False)
if "use_tc_tiling_on_sc" in pltpu.CompilerParams.__dataclass_fields__:
    _sc_params = dataclasses.replace(_sc_params, use_tc_tiling_on_sc=False)


@functools.partial(
    pl.kernel,
    out_type=jax.ShapeDtypeStruct((NW, NBD, BB), jnp.float32),
    mesh=_mesh,
    scratch_types=[
        pltpu.VMEM((NBD, BB), jnp.int32),
        pltpu.VMEM((NBD, BB), jnp.int32),
        pltpu.VMEM((NBD, BB), jnp.float32),
        pltpu.VMEM((BB, 16), jnp.float32),
        pltpu.VMEM((BB, 16), jnp.float32),
        pltpu.SemaphoreType.DMA,
        pltpu.SemaphoreType.DMA,
    ],
    compiler_params=_sc_params,
)
def _d2_kernel(zp_hbm, src_hbm, dst_hbm, d2_hbm, srcv, dstv, d2v, zs, zd, sem1, sem2):
    cid = lax.axis_index("c")
    sid = lax.axis_index("s")
    wid = sid * NC + cid
    pltpu.sync_copy(src_hbm.at[wid], srcv)
    pltpu.sync_copy(dst_hbm.at[wid], dstv)

    @pl.loop(0, NBD)
    def _(b):
        cp1 = pltpu.async_copy(zp_hbm.at[srcv.at[b]], zs, sem1)
        cp2 = pltpu.async_copy(zp_hbm.at[dstv.at[b]], zd, sem2)
        cp1.wait()
        cp2.wait()

        @pl.loop(0, BB // 16)
        def _(g):
            rows = g * 16 + lax.iota(jnp.int32, 16)
            acc = None
            for j in range(3):
                cols = jnp.full((16,), j, jnp.int32)
                cs = plsc.load_gather(zs, [rows, cols])
                cd = plsc.load_gather(zd, [rows, cols])
                df = cd - cs
                acc = df * df if acc is None else acc + df * df
            d2v[b, pl.ds(g * 16, 16)] = acc

    pltpu.sync_copy(d2v, d2_hbm.at[wid])


@functools.partial(
    pl.kernel,
    out_type=jax.ShapeDtypeStruct((NC, N, HH), jnp.float32),
    mesh=_mesh,
    scratch_types=[
        pltpu.VMEM((NBL, BB), jnp.int32),
        pltpu.VMEM((NBL, BB), jnp.int32),
        pltpu.VMEM((BB, HH), jnp.float32),
        pltpu.VMEM((BB, HH), jnp.float32),
        pltpu.VMEM((BB, HH), jnp.float32),
        pltpu.VMEM((BB, HH), jnp.float32),
        pltpu.VMEM((BB, HH), jnp.float32),
        pltpu.VMEM((BB, HH), jnp.float32),
        pltpu.VMEM((BB, HH), jnp.float32),
        pltpu.VMEM((BB, HH), jnp.float32),
        pltpu.VMEM_SHARED((N, HH), jnp.float32),
        pltpu.SemaphoreType.DMA,
        pltpu.SemaphoreType.DMA,
        pltpu.SemaphoreType.DMA,
        pltpu.SemaphoreType.DMA,
        pltpu.SemaphoreType.DMA,
        pltpu.SemaphoreType.DMA,
        pltpu.SemaphoreType.DMA,
        pltpu.SemaphoreType.DMA,
    ],
    compiler_params=_sc_params,
)
def _layer_kernel(h_hbm, src_hbm, dst_hbm, c_hbm, zeros_hbm, agg_hbm,
                  srcv, dstv,
                  bufS0, bufS1, bufD0, bufD1, bufC0, bufC1, bufM0, bufM1, aggsh,
                  semS0, semS1, semD0, semD1, semC0, semC1, semM0, semM1):
    cid = lax.axis_index("c")
    sid = lax.axis_index("s")

    zstart = pl.multiple_of(sid * RPT, 8)

    @pl.when(sid < NS - 1)
    def _():
        pltpu.sync_copy(zeros_hbm.at[pl.ds(zstart, RPT)],
                        aggsh.at[pl.ds(zstart, RPT)])

    @pl.when(sid == NS - 1)
    def _():
        pltpu.sync_copy(zeros_hbm.at[pl.ds((NS - 1) * RPT, N - (NS - 1) * RPT)],
                        aggsh.at[pl.ds((NS - 1) * RPT, N - (NS - 1) * RPT)])

    pltpu.sync_copy(src_hbm.at[sid], srcv)
    pltpu.sync_copy(dst_hbm.at[sid], dstv)
    plsc.subcore_barrier()

    bufS = (bufS0, bufS1)
    bufD = (bufD0, bufD1)
    bufC = (bufC0, bufC1)
    bufM = (bufM0, bufM1)
    semS = (semS0, semS1)
    semD = (semD0, semD1)
    semC = (semC0, semC1)
    semM = (semM0, semM1)
    NJ = HH // 16

    def c_slice(b):
        return c_hbm.at[cid, pl.ds(pl.multiple_of(sid * EWL + b * BB, 8), BB)]

    def issue_gathers(b, p):
        pltpu.async_copy(h_hbm.at[cid].at[srcv.at[b]], bufS[p], semS[p])
        pltpu.async_copy(h_hbm.at[cid].at[dstv.at[b]], bufD[p], semD[p])
        pltpu.async_copy(c_slice(b), bufC[p], semC[p])

    def wait_gathers(b, p):
        pltpu.make_async_copy(h_hbm.at[cid].at[srcv.at[b]], bufS[p], semS[p]).wait()
        pltpu.make_async_copy(h_hbm.at[cid].at[dstv.at[b]], bufD[p], semD[p]).wait()
        pltpu.make_async_copy(c_slice(b), bufC[p], semC[p]).wait()

    def wait_scatter(b, p):
        pltpu.make_async_copy(bufM[p], aggsh.at[dstv.at[b]], semM[p]).wait()

    def compute_batch(b, p):
        bS, bD, bC, bM = bufS[p], bufD[p], bufC[p], bufM[p]

        @plsc.parallel_loop(0, BB, unroll=4)
        def _(e):
            for j in range(NJ):
                o = j * 16
                m = (bS[e, pl.ds(o, 16)] + bD[e, pl.ds(o, 16)]
                     + bC[e, pl.ds(o, 16)])
                bM[e, pl.ds(o, 16)] = m / (1.0 + jnp.exp(-m))

    issue_gathers(0, 0)
    issue_gathers(1, 1)

    @pl.loop(0, NBL // 2)
    def _(i):
        for p in range(2):
            b = 2 * i + p
            wait_gathers(b, p)
            compute_batch(b, p)

            @pl.when(i < NBL // 2 - 1)
            def _():
                issue_gathers(b + 2, p)

            pltpu.sync_copy(bufM[p], aggsh.at[dstv.at[b]], add=True)

    plsc.subcore_barrier()

    @pl.when(sid < NS - 1)
    def _():
        pltpu.sync_copy(aggsh.at[pl.ds(zstart, RPT)],
                        agg_hbm.at[cid, pl.ds(zstart, RPT)])

    @pl.when(sid == NS - 1)
    def _():
        pltpu.sync_copy(aggsh.at[pl.ds((NS - 1) * RPT, N - (NS - 1) * RPT)],
                        agg_hbm.at[cid, pl.ds((NS - 1) * RPT, N - (NS - 1) * RPT)])


def _cbuild_body(et_ref, d2_ref, ewt_ref, wr_ref, c_ref):
    onehot = (et_ref[...] == lax.broadcasted_iota(jnp.int32, (1, 4), 1))
    c = (jnp.dot(onehot.astype(jnp.float32), ewt_ref[...],
                 preferred_element_type=jnp.float32)
         + d2_ref[...] * wr_ref[...])
    c_ref[0] = c[:, :HH]
    c_ref[1] = c[:, HH:]


def _prep_body(B_ref, bemb_ref, eemb_ref, We_ref, h0_ref, ewt_ref):
    i = pl.program_id(0)
    onehot = (B_ref[...] == lax.broadcasted_iota(jnp.int32, (1, VOC), 1))
    h0 = jnp.dot(onehot.astype(jnp.float32), bemb_ref[...],
                 preferred_element_type=jnp.float32)
    h0_ref[0] = h0[:, :HH]
    h0_ref[1] = h0[:, HH:]

    @pl.when(i == 0)
    def _():
        ewt_ref[...] = jnp.dot(eemb_ref[...], We_ref[...],
                               preferred_element_type=jnp.float32)


def _upd_body(agg_ref, h_ref, Wh_ref, out_ref):
    a = jnp.concatenate([agg_ref[0], agg_ref[1]], axis=1)
    h = jnp.concatenate([h_ref[0], h_ref[1]], axis=1)
    y = jnp.dot(a, Wh_ref[...], preferred_element_type=jnp.float32)
    res = h + y / (1.0 + jnp.exp(-y))
    out_ref[0] = res[:, :HH]
    out_ref[1] = res[:, HH:]


def _ffn_body(h_ref, bid_ref, W1_ref, W2_ref, en_ref):
    i = pl.program_id(0)
    h = jnp.concatenate([h_ref[0], h_ref[1]], axis=1)
    x = h / (1.0 + jnp.exp(-h))
    y = jnp.dot(x, W1_ref[...], preferred_element_type=jnp.float32)
    y = y / (1.0 + jnp.exp(-y))
    ne = jnp.dot(y, W2_ref[...], preferred_element_type=jnp.float32)
    mask = (bid_ref[...] == lax.broadcasted_iota(jnp.int32, (1, G), 1))
    part = jnp.sum(mask.astype(jnp.float32) * ne, axis=0, keepdims=True)

    @pl.when(i == 0)
    def _():
        en_ref[...] = jnp.zeros_like(en_ref)

    en_ref[...] += part


def kernel(Z, B, edge_index, edge_type, batch_id, block_emb, edge_emb, We, wr, Wh, W1, W2):
    src = edge_index[0].astype(jnp.int32)
    dst = edge_index[1].astype(jnp.int32)
    et = edge_type.astype(jnp.int32)
    src_d = src.reshape(NW, NBD, BB)
    dst_d = dst.reshape(NW, NBD, BB)
    src_l = src.reshape(NS, NBL, BB)
    dst_l = dst.reshape(NS, NBL, BB)
    zp = jnp.pad(Z.astype(jnp.float32), ((0, 0), (0, 13)))
    zeros = jnp.zeros((N, HH), jnp.float32)
    Bi = B.astype(jnp.int32).reshape(N, 1)
    bid = batch_id.astype(jnp.int32).reshape(N, 1)

    h0, ewt = pl.pallas_call(
        _prep_body,
        out_shape=(jax.ShapeDtypeStruct((NC, N, HH), jnp.float32),
                   jax.ShapeDtypeStruct((4, H), jnp.float32)),
        grid=(N // BLK,),
        in_specs=[
            pl.BlockSpec((BLK, 1), lambda i: (i, 0)),
            pl.BlockSpec((VOC, H), lambda i: (0, 0)),
            pl.BlockSpec((4, 64), lambda i: (0, 0)),
            pl.BlockSpec((64, H), lambda i: (0, 0)),
        ],
        out_specs=(pl.BlockSpec((NC, BLK, HH), lambda i: (0, i, 0)),
                   pl.BlockSpec((4, H), lambda i: (0, 0))),
    )(Bi, block_emb, edge_emb, We)

    d2 = _d2_kernel(zp, src_d, dst_d)

    EBLK = 2000
    c = pl.pallas_call(
        _cbuild_body,
        out_shape=jax.ShapeDtypeStruct((NC, E, HH), jnp.float32),
        grid=(E // EBLK,),
        in_specs=[
            pl.BlockSpec((EBLK, 1), lambda i: (i, 0)),
            pl.BlockSpec((EBLK, 1), lambda i: (i, 0)),
            pl.BlockSpec((4, H), lambda i: (0, 0)),
            pl.BlockSpec((1, H), lambda i: (0, 0)),
        ],
        out_specs=pl.BlockSpec((NC, EBLK, HH), lambda i: (0, i, 0)),
    )(et.reshape(E, 1), d2.reshape(E, 1), ewt, wr.astype(jnp.float32).reshape(1, H))

    h = h0
    for _ in range(3):
        agg = _layer_kernel(h, src_l, dst_l, c, zeros)
        h = pl.pallas_call(
            _upd_body,
            out_shape=jax.ShapeDtypeStruct((NC, N, HH), jnp.float32),
            grid=(N // BLK,),
            in_specs=[
                pl.BlockSpec((NC, BLK, HH), lambda i: (0, i, 0)),
                pl.BlockSpec((NC, BLK, HH), lambda i: (0, i, 0)),
                pl.BlockSpec((H, H), lambda i: (0, 0)),
            ],
            out_specs=pl.BlockSpec((NC, BLK, HH), lambda i: (0, i, 0)),
        )(agg, h, Wh)

    en = pl.pallas_call(
        _ffn_body,
        out_shape=jax.ShapeDtypeStruct((1, G), jnp.float32),
        grid=(N // BLK,),
        in_specs=[
            pl.BlockSpec((NC, BLK, HH), lambda i: (0, i, 0)),
            pl.BlockSpec((BLK, 1), lambda i: (i, 0)),
            pl.BlockSpec((H, H), lambda i: (0, 0)),
            pl.BlockSpec((H, 1), lambda i: (0, 0)),
        ],
        out_specs=pl.BlockSpec((1, G), lambda i: (0, 0)),
    )(h, bid, W1, W2)
    return en[0]


# speedup vs baseline: 1.1855x; 1.1855x over previous
import dataclasses
import functools

import jax
import jax.numpy as jnp
from jax import lax
from jax.experimental import pallas as pl
from jax.experimental.pallas import tpu as pltpu
from jax.experimental.pallas import tpu_sc as plsc

N = 10000
E = 320000
H = 128
HH = H // 2
G = 32
VOC = 100
NC = 2
NS = 16
NW = NC * NS
BB = 80
EWD = E // NW
NBD = EWD // BB
EWL = E // NS
NBL = EWL // BB
RPT = 624
BLK = 1000

_mesh = plsc.VectorSubcoreMesh(core_axis_name="c", subcore_axis_name="s")

_sc_params = pltpu.CompilerParams()
if "needs_layout_passes" in pltpu.CompilerParams.__dataclass_fields__:
    _sc_params = dataclasses.replace(_sc_params, needs_layout_passes=False)
if "use_tc_tiling_on_sc" in pltpu.CompilerParams.__dataclass_fields__:
    _sc_params = dataclasses.replace(_sc_params, use_tc_tiling_on_sc=False)


@functools.partial(
    pl.kernel,
    out_type=jax.ShapeDtypeStruct((NC, E, HH), jnp.float32),
    mesh=_mesh,
    scratch_types=[
        pltpu.VMEM((NBD, BB), jnp.int32),
        pltpu.VMEM((NBD, BB), jnp.int32),
        pltpu.VMEM((NBD, BB), jnp.int32),
        pltpu.VMEM((NC, 4 * HH), jnp.float32),
        pltpu.VMEM((NC, HH), jnp.float32),
        pltpu.VMEM((BB, 16), jnp.float32),
        pltpu.VMEM((BB, 16), jnp.float32),
        pltpu.VMEM((BB, 16), jnp.float32),
        pltpu.VMEM((BB, 16), jnp.float32),
        pltpu.VMEM((NC, BB, HH), jnp.float32),
        pltpu.VMEM((NC, BB, HH), jnp.float32),
        pltpu.SemaphoreType.DMA,
        pltpu.SemaphoreType.DMA,
        pltpu.SemaphoreType.DMA,
        pltpu.SemaphoreType.DMA,
        pltpu.SemaphoreType.DMA,
        pltpu.SemaphoreType.DMA,
    ],
    compiler_params=_sc_params,
)
def _cgen_kernel(zp_hbm, src_hbm, dst_hbm, et_hbm, ewt_hbm, wr_hbm, c_hbm,
                 srcv, dstv, etv, ewtv, wrv,
                 zs0, zs1, zd0, zd1, cb0, cb1,
                 semS0, semS1, semD0, semD1, semC0, semC1):
    cid = lax.axis_index("c")
    sid = lax.axis_index("s")
    wid = sid * NC + cid
    pltpu.sync_copy(src_hbm.at[wid], srcv)
    pltpu.sync_copy(dst_hbm.at[wid], dstv)
    pltpu.sync_copy(et_hbm.at[wid], etv)
    pltpu.sync_copy(ewt_hbm, ewtv)
    pltpu.sync_copy(wr_hbm, wrv)

    zs = (zs0, zs1)
    zd = (zd0, zd1)
    cb = (cb0, cb1)
    semS = (semS0, semS1)
    semD = (semD0, semD1)
    semC = (semC0, semC1)
    wrs = [[wrv[c_, pl.ds(j * 16, 16)] for j in range(HH // 16)]
           for c_ in range(NC)]

    def issue_gathers(b, p):
        pltpu.async_copy(zp_hbm.at[srcv.at[b]], zs[p], semS[p])
        pltpu.async_copy(zp_hbm.at[dstv.at[b]], zd[p], semD[p])

    def wait_gathers(b, p):
        pltpu.make_async_copy(zp_hbm.at[srcv.at[b]], zs[p], semS[p]).wait()
        pltpu.make_async_copy(zp_hbm.at[dstv.at[b]], zd[p], semD[p]).wait()

    def c_out(b, c_):
        off = pl.multiple_of(wid * EWD + b * BB, 8)
        return c_hbm.at[c_, pl.ds(off, BB)]

    def compute_batch(b, p):
        @pl.loop(0, BB // 16)
        def _(g):
            rows = g * 16 + lax.iota(jnp.int32, 16)
            acc = None
            for j in range(3):
                cols = jnp.full((16,), j, jnp.int32)
                cs = plsc.load_gather(zs[p], [rows, cols])
                cd = plsc.load_gather(zd[p], [rows, cols])
                df = cd - cs
                acc = df * df if acc is None else acc + df * df
            et_vec = etv[b, pl.ds(g * 16, 16)]
            for e16 in range(16):
                e = g * 16 + e16
                t = et_vec[e16]
                d2s = acc[e16]
                base = t * HH
                for c_ in range(NC):
                    for j in range(HH // 16):
                        o = j * 16
                        cb[p][c_, e, pl.ds(o, 16)] = (
                            ewtv[c_, pl.ds(pl.multiple_of(base + o, 16), 16)]
                            + d2s * wrs[c_][j])

    issue_gathers(0, 0)
    issue_gathers(1, 1)

    @pl.loop(0, NBD // 2)
    def _(i):
        for p in range(2):
            b = 2 * i + p

            @pl.when(i > 0)
            def _():
                for c_ in range(NC):
                    pltpu.make_async_copy(cb[p].at[c_], c_out(b - 2, c_),
                                          semC[p]).wait()

            wait_gathers(b, p)
            compute_batch(b, p)

            @pl.when(i < NBD // 2 - 1)
            def _():
                issue_gathers(b + 2, p)

            for c_ in range(NC):
                pltpu.async_copy(cb[p].at[c_], c_out(b, c_), semC[p])

    for p in range(2):
        for c_ in range(NC):
            pltpu.make_async_copy(cb[p].at[c_], c_out(NBD - 2 + p, c_),
                                  semC[p]).wait()


@functools.partial(
    pl.kernel,
    out_type=jax.ShapeDtypeStruct((NC, N, HH), jnp.float32),
    mesh=_mesh,
    scratch_types=[
        pltpu.VMEM((NBL, BB), jnp.int32),
        pltpu.VMEM((NBL, BB), jnp.int32),
        pltpu.VMEM((BB, HH), jnp.float32),
        pltpu.VMEM((BB, HH), jnp.float32),
        pltpu.VMEM((BB, HH), jnp.float32),
        pltpu.VMEM((BB, HH), jnp.float32),
        pltpu.VMEM((BB, HH), jnp.float32),
        pltpu.VMEM((BB, HH), jnp.float32),
        pltpu.VMEM((BB, HH), jnp.float32),
        pltpu.VMEM((BB, HH), jnp.float32),
        pltpu.VMEM_SHARED((N, HH), jnp.float32),
        pltpu.SemaphoreType.DMA,
        pltpu.SemaphoreType.DMA,
        pltpu.SemaphoreType.DMA,
        pltpu.SemaphoreType.DMA,
        pltpu.SemaphoreType.DMA,
        pltpu.SemaphoreType.DMA,
        pltpu.SemaphoreType.DMA,
        pltpu.SemaphoreType.DMA,
    ],
    compiler_params=_sc_params,
)
def _layer_kernel(h_hbm, src_hbm, dst_hbm, c_hbm, zeros_hbm, agg_hbm,
                  srcv, dstv,
                  bufS0, bufS1, bufD0, bufD1, bufC0, bufC1, bufM0, bufM1, aggsh,
                  semS0, semS1, semD0, semD1, semC0, semC1, semM0, semM1):
    cid = lax.axis_index("c")
    sid = lax.axis_index("s")

    zstart = pl.multiple_of(sid * RPT, 8)

    @pl.when(sid < NS - 1)
    def _():
        pltpu.sync_copy(zeros_hbm.at[pl.ds(zstart, RPT)],
                        aggsh.at[pl.ds(zstart, RPT)])

    @pl.when(sid == NS - 1)
    def _():
        pltpu.sync_copy(zeros_hbm.at[pl.ds((NS - 1) * RPT, N - (NS - 1) * RPT)],
                        aggsh.at[pl.ds((NS - 1) * RPT, N - (NS - 1) * RPT)])

    pltpu.sync_copy(src_hbm.at[sid], srcv)
    pltpu.sync_copy(dst_hbm.at[sid], dstv)
    plsc.subcore_barrier()

    bufS = (bufS0, bufS1)
    bufD = (bufD0, bufD1)
    bufC = (bufC0, bufC1)
    bufM = (bufM0, bufM1)
    semS = (semS0, semS1)
    semD = (semD0, semD1)
    semC = (semC0, semC1)
    semM = (semM0, semM1)
    NJ = HH // 16

    def c_slice(b):
        return c_hbm.at[cid, pl.ds(pl.multiple_of(sid * EWL + b * BB, 8), BB)]

    def issue_gathers(b, p):
        pltpu.async_copy(h_hbm.at[cid].at[srcv.at[b]], bufS[p], semS[p])
        pltpu.async_copy(h_hbm.at[cid].at[dstv.at[b]], bufD[p], semD[p])
        pltpu.async_copy(c_slice(b), bufC[p], semC[p])

    def wait_gathers(b, p):
        pltpu.make_async_copy(h_hbm.at[cid].at[srcv.at[b]], bufS[p], semS[p]).wait()
        pltpu.make_async_copy(h_hbm.at[cid].at[dstv.at[b]], bufD[p], semD[p]).wait()
        pltpu.make_async_copy(c_slice(b), bufC[p], semC[p]).wait()

    def wait_scatter(b, p):
        pltpu.make_async_copy(bufM[p], aggsh.at[dstv.at[b]], semM[p]).wait()

    def compute_batch(b, p):
        bS, bD, bC, bM = bufS[p], bufD[p], bufC[p], bufM[p]

        @pl.loop(0, BB)
        def _(e):
            for j in range(NJ):
                o = j * 16
                m = (bS[e, pl.ds(o, 16)] + bD[e, pl.ds(o, 16)]
                     + bC[e, pl.ds(o, 16)])
                bM[e, pl.ds(o, 16)] = m / (1.0 + jnp.exp(-m))

    issue_gathers(0, 0)
    issue_gathers(1, 1)

    @pl.loop(0, NBL // 2)
    def _(i):
        for p in range(2):
            b = 2 * i + p
            wait_gathers(b, p)
            compute_batch(b, p)

            @pl.when(i < NBL // 2 - 1)
            def _():
                issue_gathers(b + 2, p)

            pltpu.sync_copy(bufM[p], aggsh.at[dstv.at[b]], add=True)

    plsc.subcore_barrier()

    @pl.when(sid < NS - 1)
    def _():
        pltpu.sync_copy(aggsh.at[pl.ds(zstart, RPT)],
                        agg_hbm.at[cid, pl.ds(zstart, RPT)])

    @pl.when(sid == NS - 1)
    def _():
        pltpu.sync_copy(aggsh.at[pl.ds((NS - 1) * RPT, N - (NS - 1) * RPT)],
                        agg_hbm.at[cid, pl.ds((NS - 1) * RPT, N - (NS - 1) * RPT)])


def _cbuild_body(et_ref, d2_ref, ewt_ref, wr_ref, c_ref):
    onehot = (et_ref[...] == lax.broadcasted_iota(jnp.int32, (1, 4), 1))
    c = (jnp.dot(onehot.astype(jnp.float32), ewt_ref[...],
                 preferred_element_type=jnp.float32)
         + d2_ref[...] * wr_ref[...])
    c_ref[0] = c[:, :HH]
    c_ref[1] = c[:, HH:]


def _prep_body(B_ref, bemb_ref, eemb_ref, We_ref, h0_ref, ewt_ref):
    i = pl.program_id(0)
    onehot = (B_ref[...] == lax.broadcasted_iota(jnp.int32, (1, VOC), 1))
    h0 = jnp.dot(onehot.astype(jnp.float32), bemb_ref[...],
                 preferred_element_type=jnp.float32)
    h0_ref[0] = h0[:, :HH]
    h0_ref[1] = h0[:, HH:]

    @pl.when(i == 0)
    def _():
        ewt_ref[...] = jnp.dot(eemb_ref[...], We_ref[...],
                               preferred_element_type=jnp.float32)


def _upd_body(agg_ref, h_ref, Wh_ref, out_ref):
    a = jnp.concatenate([agg_ref[0], agg_ref[1]], axis=1)
    h = jnp.concatenate([h_ref[0], h_ref[1]], axis=1)
    y = jnp.dot(a, Wh_ref[...], preferred_element_type=jnp.float32)
    res = h + y / (1.0 + jnp.exp(-y))
    out_ref[0] = res[:, :HH]
    out_ref[1] = res[:, HH:]


def _ffn_body(h_ref, bid_ref, W1_ref, W2_ref, en_ref):
    i = pl.program_id(0)
    h = jnp.concatenate([h_ref[0], h_ref[1]], axis=1)
    x = h / (1.0 + jnp.exp(-h))
    y = jnp.dot(x, W1_ref[...], preferred_element_type=jnp.float32)
    y = y / (1.0 + jnp.exp(-y))
    ne = jnp.dot(y, W2_ref[...], preferred_element_type=jnp.float32)
    mask = (bid_ref[...] == lax.broadcasted_iota(jnp.int32, (1, G), 1))
    part = jnp.sum(mask.astype(jnp.float32) * ne, axis=0, keepdims=True)

    @pl.when(i == 0)
    def _():
        en_ref[...] = jnp.zeros_like(en_ref)

    en_ref[...] += part


def kernel(Z, B, edge_index, edge_type, batch_id, block_emb, edge_emb, We, wr, Wh, W1, W2):
    src = edge_index[0].astype(jnp.int32)
    dst = edge_index[1].astype(jnp.int32)
    et = edge_type.astype(jnp.int32)
    src_d = src.reshape(NW, NBD, BB)
    dst_d = dst.reshape(NW, NBD, BB)
    src_l = src.reshape(NS, NBL, BB)
    dst_l = dst.reshape(NS, NBL, BB)
    zp = jnp.pad(Z.astype(jnp.float32), ((0, 0), (0, 13)))
    zeros = jnp.zeros((N, HH), jnp.float32)
    Bi = B.astype(jnp.int32).reshape(N, 1)
    bid = batch_id.astype(jnp.int32).reshape(N, 1)

    h0, ewt = pl.pallas_call(
        _prep_body,
        out_shape=(jax.ShapeDtypeStruct((NC, N, HH), jnp.float32),
                   jax.ShapeDtypeStruct((4, H), jnp.float32)),
        grid=(N // BLK,),
        in_specs=[
            pl.BlockSpec((BLK, 1), lambda i: (i, 0)),
            pl.BlockSpec((VOC, H), lambda i: (0, 0)),
            pl.BlockSpec((4, 64), lambda i: (0, 0)),
            pl.BlockSpec((64, H), lambda i: (0, 0)),
        ],
        out_specs=(pl.BlockSpec((NC, BLK, HH), lambda i: (0, i, 0)),
                   pl.BlockSpec((4, H), lambda i: (0, 0))),
    )(Bi, block_emb, edge_emb, We)

    ewt_split = ewt.reshape(4, NC, HH).transpose(1, 0, 2).reshape(NC, 4 * HH)
    wr2 = wr.astype(jnp.float32).reshape(NC, HH)
    et_d = et.reshape(NW, NBD, BB)
    c = _cgen_kernel(zp, src_d, dst_d, et_d, ewt_split, wr2)

    h = h0
    for _ in range(3):
        agg = _layer_kernel(h, src_l, dst_l, c, zeros)
        h = pl.pallas_call(
            _upd_body,
            out_shape=jax.ShapeDtypeStruct((NC, N, HH), jnp.float32),
            grid=(N // BLK,),
            in_specs=[
                pl.BlockSpec((NC, BLK, HH), lambda i: (0, i, 0)),
                pl.BlockSpec((NC, BLK, HH), lambda i: (0, i, 0)),
                pl.BlockSpec((H, H), lambda i: (0, 0)),
            ],
            out_specs=pl.BlockSpec((NC, BLK, HH), lambda i: (0, i, 0)),
        )(agg, h, Wh)

    en = pl.pallas_call(
        _ffn_body,
        out_shape=jax.ShapeDtypeStruct((1, G), jnp.float32),
        grid=(N // BLK,),
        in_specs=[
            pl.BlockSpec((NC, BLK, HH), lambda i: (0, i, 0)),
            pl.BlockSpec((BLK, 1), lambda i: (i, 0)),
            pl.BlockSpec((H, H), lambda i: (0, 0)),
            pl.BlockSpec((H, 1), lambda i: (0, 0)),
        ],
        out_specs=pl.BlockSpec((1, G), lambda i: (0, 0)),
    )(h, bid, W1, W2)
    return en[0]
